# baseline (speedup 1.0000x reference)
import jax
import jax.numpy as jnp
from jax import lax
from jax.experimental import pallas as pl
from jax.experimental.pallas import tpu as pltpu
from jax.experimental.pallas import tpu_sc as plsc


_C = 1024
_K = 12

_NW = 32
_L = 16


def _in_copy(z_hbm, zbuf, insem, chunk, slot):
    return pltpu.make_async_copy(
        z_hbm.at[pl.ds(chunk * _C, _C), :], zbuf.at[slot], insem.at[slot]
    )


def _out_copy(obuf, o_hbm, outsem, chunk, slot):
    nb = _C // 128
    return pltpu.make_async_copy(
        obuf.at[slot], o_hbm.at[pl.ds(chunk * nb, nb)], outsem.at[slot]
    )


def _matmul_body(z_hbm, w_ref, b_ref, o_hbm, zbuf, obuf, insem, outsem):
    n_chunks = z_hbm.shape[0] // _C
    nb = _C // 128
    w = w_ref[...]
    b = b_ref[...]

    for s in range(_K):
        _in_copy(z_hbm, zbuf, insem, s, s).start()

    def step(i, carry):
        slot = jax.lax.rem(i, _K)
        _in_copy(z_hbm, zbuf, insem, i, slot).wait()

        @pl.when(i >= _K)
        def _():
            _out_copy(obuf, o_hbm, outsem, i - _K, slot).wait()

        z = zbuf[slot]
        logits = jax.lax.dot_general(
            z, w, (((1,), (0,)), ((), ())), preferred_element_type=jnp.float32
        )
        lt = jnp.transpose(logits) + b
        obuf[slot] = jnp.transpose(lt.reshape(8, nb, 128), (1, 0, 2))
        _out_copy(obuf, o_hbm, outsem, i, slot).start()

        @pl.when(i + _K < n_chunks)
        def _():
            _in_copy(z_hbm, zbuf, insem, i + _K, slot).start()

        return carry

    jax.lax.fori_loop(0, n_chunks, step, 0)

    for s in range(_K):
        chunk = n_chunks - _K + s
        _out_copy(obuf, o_hbm, outsem, chunk, chunk % _K).wait()


def _sc_softmax_body(lt_hbm, o_hbm, lbuf, obuf):
    n_blocks = lt_hbm.shape[0]
    bpw = n_blocks // _NW
    tok_pw = bpw * 128
    wid = lax.axis_index("s") * 2 + lax.axis_index("c")
    pltpu.sync_copy(lt_hbm.at[pl.ds(wid * bpw, bpw)], lbuf)
    lane = lax.iota(jnp.int32, _L)
    for blk in range(bpw):
        for grp in range(128 // _L):
            off = grp * _L
            vecs = [lbuf[blk, e, pl.ds(off, _L)] for e in range(8)]
            m = vecs[0]
            for e in range(1, 8):
                m = jnp.maximum(m, vecs[e])
            exps = [jnp.exp(v - m) for v in vecs]
            tot = exps[0]
            for e in range(1, 8):
                tot = tot + exps[e]
            inv = 1.0 / tot
            base = (blk * 128 + off) * 8
            for e in range(8):
                plsc.store_scatter(
                    obuf, [base + e + lane * 8], exps[e] * inv
                )
    pltpu.sync_copy(
        obuf, o_hbm.at[pl.ds(wid * tok_pw * 8, tok_pw * 8)]
    )


def _sc_softmax(lt3, n_tokens, n_exp):
    n_blocks = lt3.shape[0]
    bpw = n_blocks // _NW
    mesh = plsc.VectorSubcoreMesh(core_axis_name="c", subcore_axis_name="s")

    def body(lt_hbm, o_hbm, lbuf, obuf):
        _sc_softmax_body(lt_hbm, o_hbm, lbuf, obuf)

    return pl.kernel(
        body,
        mesh=mesh,
        out_type=jax.ShapeDtypeStruct((n_tokens * n_exp,), jnp.float32),
        scratch_types=[
            pltpu.VMEM((bpw, 8, 128), jnp.float32),
            pltpu.VMEM((bpw * 128 * 8,), jnp.float32),
        ],
        compiler_params=pltpu.CompilerParams(needs_layout_passes=False),
    )(lt3)


@jax.jit
def kernel(z, W, b):
    n_tokens, d_model = z.shape
    n_exp = W.shape[1]
    n_blocks = n_tokens // 128
    lt3 = pl.pallas_call(
        _matmul_body,
        in_specs=[
            pl.BlockSpec(memory_space=pl.ANY),
            pl.BlockSpec(memory_space=pltpu.VMEM),
            pl.BlockSpec(memory_space=pltpu.VMEM),
        ],
        out_specs=pl.BlockSpec(memory_space=pl.ANY),
        out_shape=jax.ShapeDtypeStruct((n_blocks, n_exp, 128), jnp.float32),
        scratch_shapes=[
            pltpu.VMEM((_K, _C, d_model), jnp.float32),
            pltpu.VMEM((_K, _C // 128, n_exp, 128), jnp.float32),
            pltpu.SemaphoreType.DMA((_K,)),
            pltpu.SemaphoreType.DMA((_K,)),
        ],
    )(z, W, b.reshape(n_exp, 1))
    return _sc_softmax(lt3, n_tokens, n_exp).reshape(n_tokens, n_exp)


# speedup vs baseline: 2.2121x; 2.2121x over previous
import jax
import jax.numpy as jnp
from jax.experimental import pallas as pl
from jax.experimental.pallas import tpu as pltpu


_C = 1024
_K = 16


def _in_copy(z_hbm, zbuf, insem, chunk, slot):
    return pltpu.make_async_copy(
        z_hbm.at[pl.ds(chunk * _C, _C), :], zbuf.at[slot], insem.at[slot]
    )


def _out_copy(obuf, o_hbm, outsem, chunk, slot):
    return pltpu.make_async_copy(
        obuf.at[slot], o_hbm.at[:, pl.ds(chunk * _C, _C)], outsem.at[slot]
    )


def _gate_body(z_hbm, w_ref, b_ref, o_hbm, zbuf, obuf, insem, outsem):
    n_chunks = z_hbm.shape[0] // _C
    w = w_ref[...]
    b = b_ref[...]

    for s in range(_K):
        _in_copy(z_hbm, zbuf, insem, s, s).start()

    def step(i, carry):
        slot = jax.lax.rem(i, _K)
        _in_copy(z_hbm, zbuf, insem, i, slot).wait()

        @pl.when(i >= _K)
        def _():
            _out_copy(obuf, o_hbm, outsem, i - _K, slot).wait()

        z = zbuf[slot]
        logits = jax.lax.dot_general(
            z, w, (((1,), (0,)), ((), ())), preferred_element_type=jnp.float32
        )
        lt = jnp.transpose(logits) + b
        m = jnp.max(lt, axis=0, keepdims=True)
        e = jnp.exp(lt - m)
        obuf[slot] = e / jnp.sum(e, axis=0, keepdims=True)
        _out_copy(obuf, o_hbm, outsem, i, slot).start()

        @pl.when(i + _K < n_chunks)
        def _():
            _in_copy(z_hbm, zbuf, insem, i + _K, slot).start()

        return carry

    jax.lax.fori_loop(0, n_chunks, step, 0)

    for s in range(_K):
        chunk = n_chunks - _K + s
        _out_copy(obuf, o_hbm, outsem, chunk, chunk % _K).wait()


@jax.jit
def kernel(z, W, b):
    n_tokens, d_model = z.shape
    n_exp = W.shape[1]
    out_t = pl.pallas_call(
        _gate_body,
        in_specs=[
            pl.BlockSpec(memory_space=pl.ANY),
            pl.BlockSpec(memory_space=pltpu.VMEM),
            pl.BlockSpec(memory_space=pltpu.VMEM),
        ],
        out_specs=pl.BlockSpec(memory_space=pl.ANY),
        out_shape=jax.ShapeDtypeStruct((n_exp, n_tokens), jnp.float32),
        scratch_shapes=[
            pltpu.VMEM((_K, _C, d_model), jnp.float32),
            pltpu.VMEM((_K, n_exp, _C), jnp.float32),
            pltpu.SemaphoreType.DMA((_K,)),
            pltpu.SemaphoreType.DMA((_K,)),
        ],
    )(z, W, b.reshape(n_exp, 1))
    return out_t.T
